# baseline (speedup 1.0000x reference)
import functools

import jax
import jax.numpy as jnp
from jax.experimental import pallas as pl
from jax.experimental.pallas import tpu as pltpu


def _reduce_body(x_hbm, o_ref, buf, sem0, sem1, *, n, rows, c, scale):
    sems = (sem0, sem1)

    def copy(i, b):
        return pltpu.make_async_copy(
            x_hbm.at[pl.ds(i, 1), :, :], buf.at[pl.ds(b, 1)], sems[b])

    handles = [copy(0, 0), copy(1, 1)]
    handles[0].start()
    handles[1].start()
    for i in range(n):
        b = i % 2
        handles[b].wait()
        o_ref[pl.ds(i, 1), :, :] = (
            jnp.sum(buf[pl.ds(b, 1), :, :], axis=1, keepdims=True) * scale)
        if i + 2 < n:
            handles[b] = copy(i + 2, b)
            handles[b].start()


def kernel(x, protos):
    N, M, C, T, V = x.shape
    P = protos.shape[0]
    scale = 1.0 / (M * P)
    rows = M * V * T

    xt = jnp.transpose(x, (0, 1, 4, 3, 2)).reshape(N, rows, C)

    out = pl.pallas_call(
        functools.partial(_reduce_body, n=N, rows=rows, c=C, scale=scale),
        in_specs=[pl.BlockSpec(memory_space=pltpu.MemorySpace.HBM)],
        out_specs=pl.BlockSpec(memory_space=pltpu.MemorySpace.VMEM),
        out_shape=jax.ShapeDtypeStruct((N, 1, C), x.dtype),
        scratch_shapes=[
            pltpu.VMEM((2, rows, C), jnp.float32),
            pltpu.SemaphoreType.DMA,
            pltpu.SemaphoreType.DMA,
        ],
    )(xt)
    return out.reshape(N, C)


# speedup vs baseline: 1.2030x; 1.2030x over previous
import functools

import jax
import jax.numpy as jnp
from jax.experimental import pallas as pl
from jax.experimental.pallas import tpu as pltpu


def _reduce_body(x_hbm, o_ref, buf, sem0, sem1, sem2, sem3, *, n, rows, c, scale):
    sems = (sem0, sem1, sem2, sem3)

    def copy(i, b):
        return pltpu.make_async_copy(
            x_hbm.at[pl.ds(i, 1), :, :], buf.at[pl.ds(b, 1)], sems[b])

    nbuf = 4
    handles = [copy(i, i) for i in range(nbuf - 1)] + [None]
    for h in handles[: nbuf - 1]:
        h.start()
    for i in range(n):
        b = i % nbuf
        if i + nbuf - 1 < n:
            nb = (i + nbuf - 1) % nbuf
            handles[nb] = copy(i + nbuf - 1, nb)
            handles[nb].start()
        handles[b].wait()
        o_ref[pl.ds(i, 1), :, :] = (
            jnp.sum(buf[pl.ds(b, 1), :, :], axis=1, keepdims=True) * scale)


def kernel(x, protos):
    N, M, C, T, V = x.shape
    P = protos.shape[0]
    scale = 1.0 / (M * P)
    rows = M * V * T

    xt = jnp.transpose(x, (0, 1, 4, 3, 2)).reshape(N, rows, C)

    out = pl.pallas_call(
        functools.partial(_reduce_body, n=N, rows=rows, c=C, scale=scale),
        in_specs=[pl.BlockSpec(memory_space=pltpu.MemorySpace.HBM)],
        out_specs=pl.BlockSpec(memory_space=pltpu.MemorySpace.VMEM),
        out_shape=jax.ShapeDtypeStruct((N, 1, C), x.dtype),
        scratch_shapes=[
            pltpu.VMEM((4, rows, C), jnp.float32),
            pltpu.SemaphoreType.DMA,
            pltpu.SemaphoreType.DMA,
            pltpu.SemaphoreType.DMA,
            pltpu.SemaphoreType.DMA,
        ],
    )(xt)
    return out.reshape(N, C)
